# baseline (speedup 1.0000x reference)
import functools

import jax
import jax.numpy as jnp
from jax import lax
from jax.experimental import pallas as pl
from jax.experimental.pallas import tpu as pltpu
from jax.experimental.pallas import tpu_sc as plsc

_BATCH = 4096
_HIST = 200
_EMB = 64
_NW = 32
_BPW = _BATCH // _NW
_C0 = 104
_C1 = _HIST - _C0
_NBUF = 2
_GRP = _BPW // _NBUF

_mesh = plsc.VectorSubcoreMesh(core_axis_name="c", subcore_axis_name="s")


@functools.partial(
    pl.kernel,
    mesh=_mesh,
    out_type=jax.ShapeDtypeStruct((_BATCH // 2, 2 * _EMB), jnp.float32),
    scratch_types=[
        pltpu.VMEM((_BPW * _HIST + 16,), jnp.int32),
        pltpu.VMEM((_BPW * _HIST,), jnp.int32),
        pltpu.VMEM((_NBUF, _HIST, 2 * _EMB), jnp.float32),
        pltpu.VMEM((_BPW // 2, 2 * _EMB), jnp.float32),
        pltpu.SemaphoreType.DMA((_NBUF,)),
    ],
    compiler_params=pltpu.CompilerParams(use_tc_tiling_on_sc=False),
)
def _sum_embed(idx_hbm, table_hbm, out_hbm, idx_v, p_v, bufs, out_v, sems):
    wid = lax.axis_index("s") * 2 + lax.axis_index("c")
    base = wid * _BPW
    pltpu.sync_copy(
        idx_hbm.at[pl.ds(base * _HIST, _BPW * _HIST)],
        idx_v.at[pl.ds(0, _BPW * _HIST)],
    )

    def shift_body(k, carry):
        off = k * 16
        p_v[pl.ds(off, 16)] = lax.shift_right_logical(
            idx_v[pl.ds(off, 16)], 1
        )
        return carry

    lax.fori_loop(0, _BPW * _HIST // 16, shift_body, 0)

    def fire(r, slot):
        off = r * _HIST
        pltpu.async_copy(
            table_hbm.at[p_v.at[pl.ds(off, _C0)]],
            bufs.at[slot, pl.ds(0, _C0)],
            sems.at[slot],
        )
        pltpu.async_copy(
            table_hbm.at[p_v.at[pl.ds(off + _C0, _C1)]],
            bufs.at[slot, pl.ds(_C0, _C1)],
            sems.at[slot],
        )

    def drain(r, slot):
        off = r * _HIST
        pltpu.make_async_copy(
            table_hbm.at[p_v.at[pl.ds(off, _C0)]],
            bufs.at[slot, pl.ds(0, _C0)],
            sems.at[slot],
        ).wait()
        pltpu.make_async_copy(
            table_hbm.at[p_v.at[pl.ds(off + _C0, _C1)]],
            bufs.at[slot, pl.ds(_C0, _C1)],
            sems.at[slot],
        ).wait()

    for s in range(_NBUF):
        fire(s, s)

    def group_body(g, carry):
        for s in range(_NBUF):
            r = g * _NBUF + s
            drain(r, s)

            @pl.when(r + _NBUF < _BPW)
            def _():
                fire(r + _NBUF, s)

            def acc_body(j8, accs):
                a = list(accs)
                jb = j8 * 8
                hvec = (idx_v[pl.ds(r * _HIST + jb, 16)] & 1) * 64
                for u in range(8):
                    h64 = hvec[u]
                    for d in range(4):
                        a[d] = a[d] + bufs[
                            s, jb + u, pl.ds(h64 + d * 16, 16)
                        ]
                return tuple(a)

            zero = jnp.zeros((16,), jnp.float32)
            accs = lax.fori_loop(0, _HIST // 8, acc_body, (zero,) * 4)
            for d in range(4):
                out_v[r // 2, pl.ds((r % 2) * 64 + d * 16, 16)] = accs[d]
        return carry

    lax.fori_loop(0, _GRP, group_body, 0)
    pltpu.sync_copy(out_v, out_hbm.at[pl.ds(base // 2, _BPW // 2)])


def _linear_body(s_ref, wt_ref, b_ref, o_ref):
    o_ref[...] = (
        jnp.dot(s_ref[...], wt_ref[...], preferred_element_type=jnp.float32)
        + b_ref[...]
    )


def _linear(sums, Wt, b2d):
    return pl.pallas_call(
        _linear_body,
        out_shape=jax.ShapeDtypeStruct((_BATCH, Wt.shape[1]), jnp.float32),
    )(sums, Wt, b2d)


@jax.jit
def kernel(input, emb_table, W, b):
    idx = input.reshape(-1)
    table2 = emb_table.reshape(emb_table.shape[0] // 2, 2 * _EMB)
    sums2 = _sum_embed(idx, table2)
    sums = sums2.reshape(_BATCH, _EMB)
    out = _linear(sums, W.T, b.reshape(1, -1))
    return out


# speedup vs baseline: 1.1420x; 1.1420x over previous
import functools

import jax
import jax.numpy as jnp
from jax import lax
from jax.experimental import pallas as pl
from jax.experimental.pallas import tpu as pltpu
from jax.experimental.pallas import tpu_sc as plsc

_BATCH = 4096
_HIST = 200
_EMB = 64
_NW = 32
_BPW = _BATCH // _NW
_C0 = 104
_C1 = _HIST - _C0
_NBUF = 4
_GRP = _BPW // _NBUF

_mesh = plsc.VectorSubcoreMesh(core_axis_name="c", subcore_axis_name="s")


@functools.partial(
    pl.kernel,
    mesh=_mesh,
    out_type=jax.ShapeDtypeStruct((_BATCH, _EMB), jnp.float32),
    scratch_types=[
        pltpu.VMEM((_HIST, _BPW), jnp.int32),
        pltpu.VMEM((_BPW * _HIST,), jnp.int32),
        pltpu.VMEM((_NBUF, _HIST, _EMB), jnp.float32),
        pltpu.VMEM((_BPW, _EMB), jnp.float32),
        pltpu.SemaphoreType.DMA((_NBUF,)),
    ],
    compiler_params=pltpu.CompilerParams(
        use_tc_tiling_on_sc=False, needs_layout_passes=False
    ),
)
def _sum_embed(idxT_hbm, table_hbm, out_hbm, idx_v, idx_t, bufs, out_v, sems):
    wid = lax.axis_index("s") * 2 + lax.axis_index("c")
    base = wid * _BPW
    pltpu.sync_copy(idxT_hbm.at[:, pl.ds(base, _BPW)], idx_v)

    ivec = lax.iota(jnp.int32, 16)

    def tr_body(j, carry):
        for g in range(8):
            x = idx_v[j, pl.ds(g * 16, 16)]
            dst = (ivec + g * 16) * _HIST + j
            plsc.store_scatter(idx_t, [dst], x)
        return carry

    lax.fori_loop(0, _HIST, tr_body, 0)

    def fire(r, slot):
        off = r * _HIST
        pltpu.async_copy(
            table_hbm.at[idx_t.at[pl.ds(off, _C0)]],
            bufs.at[slot, pl.ds(0, _C0)],
            sems.at[slot],
        )
        pltpu.async_copy(
            table_hbm.at[idx_t.at[pl.ds(off + _C0, _C1)]],
            bufs.at[slot, pl.ds(_C0, _C1)],
            sems.at[slot],
        )

    def drain(r, slot):
        off = r * _HIST
        pltpu.make_async_copy(
            table_hbm.at[idx_t.at[pl.ds(off, _C0)]],
            bufs.at[slot, pl.ds(0, _C0)],
            sems.at[slot],
        ).wait()
        pltpu.make_async_copy(
            table_hbm.at[idx_t.at[pl.ds(off + _C0, _C1)]],
            bufs.at[slot, pl.ds(_C0, _C1)],
            sems.at[slot],
        ).wait()

    for s in range(_NBUF):
        fire(s, s)

    def group_body(g, carry):
        for s in range(_NBUF):
            r = g * _NBUF + s
            drain(r, s)

            @pl.when(r + _NBUF < _BPW)
            def _():
                fire(r + _NBUF, s)

            def acc_body(j8, accs):
                a = list(accs)
                jb = j8 * 8
                for u in range(8):
                    for d in range(4):
                        a[d] = a[d] + bufs[s, jb + u, pl.ds(d * 16, 16)]
                return tuple(a)

            zero = jnp.zeros((16,), jnp.float32)
            accs = lax.fori_loop(0, _HIST // 8, acc_body, (zero,) * 4)
            for d in range(4):
                out_v[r, pl.ds(d * 16, 16)] = accs[d]
        return carry

    lax.fori_loop(0, _GRP, group_body, 0)
    pltpu.sync_copy(out_v, out_hbm.at[pl.ds(base, _BPW)])


def _linear_body(s_ref, wt_ref, b_ref, o_ref):
    o_ref[...] = (
        jnp.dot(s_ref[...], wt_ref[...], preferred_element_type=jnp.float32)
        + b_ref[...]
    )


def _linear(sums, Wt, b2d):
    return pl.pallas_call(
        _linear_body,
        out_shape=jax.ShapeDtypeStruct((_BATCH, Wt.shape[1]), jnp.float32),
    )(sums, Wt, b2d)


@jax.jit
def kernel(input, emb_table, W, b):
    sums = _sum_embed(input.T, emb_table)
    out = _linear(sums, W.T, b.reshape(1, -1))
    return out
